# baseline (speedup 1.0000x reference)
import functools

import jax
import jax.numpy as jnp
from jax import lax
from jax.experimental import pallas as pl
from jax.experimental.pallas import tpu as pltpu
from jax.experimental.pallas import tpu_sc as plsc

_EPS = 1e-6
_LNEPS = -13.815510557964274


def _log1p(z):
    t = z / (2.0 + z)
    t2 = t * t
    return 2.0 * t * (1.0 + t2 * (1.0 / 3.0 + t2 * (0.2 + t2 * (1.0 / 7.0
                      + t2 / 9.0))))

_info = plsc.get_sparse_core_info()
_NS, _L = _info.num_subcores, _info.num_lanes
_NW = _NS


_DN = lax.GatherDimensionNumbers(
    offset_dims=(), collapsed_slice_dims=(0,), start_index_map=(0,))


def _dg(vec, idx):
    return lax.gather(vec, idx[:, None], _DN, slice_sizes=(1,),
                      mode=lax.GatherScatterMode.PROMISE_IN_BOUNDS)


def _lane_bcast(vec, p):
    return _dg(vec, jnp.full((_L,), p, jnp.int32))


def _vsum(vec):
    iot = lax.iota(jnp.int32, _L)
    for sh in (8, 4, 2, 1):
        vec = vec + _dg(vec, iot ^ sh)
    return vec


def _sc_body(n_real, h, w, x_hbm, y_hbm, e_hbm, tab_hbm, parts_hbm, out_hbm,
             x_v, y_v, e_v, qidx_v, cch_v, lan_v, mask_v, rows_v,
             nv_v, dv_v, tmp_v, out_v, sem):
    P = qidx_v.shape[0]
    wid = lax.axis_index("s")
    base = wid * P
    pltpu.sync_copy(x_hbm.at[pl.ds(base, P)], x_v)
    pltpu.sync_copy(y_hbm.at[pl.ds(base, P)], y_v)
    pltpu.sync_copy(e_hbm.at[pl.ds(base, P)], e_v)
    iot = lax.iota(jnp.int32, _L)
    nG = P // _L
    for j in range(nG):
        sl = pl.ds(j * _L, _L)
        x = x_v[sl]
        y = y_v[sl]
        e = e_v[sl]
        n = base + j * _L + iot
        vx = (x >= 0) & (x < w)
        vy = (y >= 0) & (y < h)
        nv = n < n_real
        xs = jnp.where(vx, x, 0)
        ys = jnp.where(vy, y, 0)
        m = jnp.where(vx & vy & nv & (e > 0), 1.0, 0.0).astype(jnp.float32)
        q = jnp.where(nv, n * h + ys, n)
        qidx_v[sl] = q
        cch_v[sl] = xs >> 4
        lan_v[sl] = xs & 15
        mask_v[sl] = m
    half = P // 2
    c0 = pltpu.async_copy(tab_hbm.at[qidx_v.at[pl.ds(0, half)]],
                          rows_v.at[pl.ds(0, half)], sem)
    c1 = pltpu.async_copy(tab_hbm.at[qidx_v.at[pl.ds(half, half)]],
                          rows_v.at[pl.ds(half, half)], sem)
    c0.wait()
    c1.wait()

    num = jnp.zeros((_L,), jnp.float32)
    den = jnp.zeros((_L,), jnp.float32)
    for j in range(nG):
        sl = pl.ds(j * _L, _L)
        cch = cch_v[sl]
        lan = lan_v[sl]
        valg = jnp.zeros((_L,), jnp.float32)
        for p in range(_L):
            pp = j * _L + p
            c_s = _lane_bcast(cch, p)
            l_s = _lane_bcast(lan, p)
            acc = jnp.zeros((_L,), jnp.float32)
            for c in range(w // _L):
                ch = rows_v[pp, pl.ds(c * _L, _L)]
                eqc = (1 - jnp.minimum(jnp.abs(c_s - c), 1)).astype(
                    jnp.float32)
                acc = acc + ch * eqc
            v_b = _dg(acc, l_s)
            eqp = (1 - jnp.minimum(jnp.abs(iot - p), 1)).astype(jnp.float32)
            valg = valg + v_b * eqp
        v = valg
        ea = jnp.exp(-jnp.abs(v))
        sp = jnp.maximum(-v, 0.0) + _log1p(ea)
        z = _EPS * (1.0 + jnp.exp(-jnp.maximum(v, -40.0)))
        loss_main = sp - _log1p(z)
        ww = jnp.exp(jnp.minimum(v, 0.0))
        ratio = ww / ((1.0 + ww) * _EPS)
        loss_tail = -_LNEPS - _log1p(ratio)
        tsel = jnp.maximum(jnp.sign(-13.8 - v), 0.0)
        lossg = loss_tail * tsel + loss_main * (1.0 - tsel)
        mg = mask_v[sl]
        num = num + lossg * mg
        den = den + mg

    nv_v[...] = num
    dv_v[...] = den
    pltpu.sync_copy(nv_v, parts_hbm.at[2 * wid])
    pltpu.sync_copy(dv_v, parts_hbm.at[2 * wid + 1])
    plsc.subcore_barrier()

    @pl.when(wid == 0)
    def _final():
        pltpu.sync_copy(parts_hbm, tmp_v)
        tn = jnp.zeros((_L,), jnp.float32)
        td = jnp.zeros((_L,), jnp.float32)
        for t in range(_NW):
            tn = tn + tmp_v[2 * t]
            td = td + tmp_v[2 * t + 1]
        sn = _vsum(tn)
        sd = _vsum(td)
        out_v[...] = sn / (sd + _EPS)
        pltpu.sync_copy(out_v, out_hbm)


def kernel(feats, xyens):
    B, K, H, W = feats.shape
    N = B * K
    chunk = _NW * _L
    P = ((N + chunk - 1) // chunk) * _L
    NPAD = _NW * P

    xy = xyens.reshape(N, 3).astype(jnp.int32)
    pad = NPAD - N
    x_flat = jnp.pad(xy[:, 0], (0, pad))
    y_flat = jnp.pad(xy[:, 1], (0, pad))
    e_flat = jnp.pad(xy[:, 2], (0, pad))
    tab = feats.reshape(B * K * H, W)

    sc_call = pl.kernel(
        functools.partial(_sc_body, N, H, W),
        mesh=plsc.VectorSubcoreMesh(core_axis_name="c", subcore_axis_name="s",
                                    num_cores=1),
        out_type=[jax.ShapeDtypeStruct((2 * _NW, _L), jnp.float32),
                  jax.ShapeDtypeStruct((_L,), jnp.float32)],
        scratch_types=[
            pltpu.VMEM((P,), jnp.int32),
            pltpu.VMEM((P,), jnp.int32),
            pltpu.VMEM((P,), jnp.int32),
            pltpu.VMEM((P,), jnp.int32),
            pltpu.VMEM((P,), jnp.int32),
            pltpu.VMEM((P,), jnp.int32),
            pltpu.VMEM((P,), jnp.float32),
            pltpu.VMEM((P, W), jnp.float32),
            pltpu.VMEM((_L,), jnp.float32),
            pltpu.VMEM((_L,), jnp.float32),
            pltpu.VMEM((2 * _NW, _L), jnp.float32),
            pltpu.VMEM((_L,), jnp.float32),
            pltpu.SemaphoreType.DMA,
        ],
    )
    _, out = sc_call(x_flat, y_flat, e_flat, tab)
    return out[0]


# speedup vs baseline: 1.1918x; 1.1918x over previous
import functools

import jax
import jax.numpy as jnp
from jax import lax
from jax.experimental import pallas as pl
from jax.experimental.pallas import tpu as pltpu
from jax.experimental.pallas import tpu_sc as plsc

_EPS = 1e-6
_LNEPS = -13.815510557964274


def _log1p(z):
    t = z / (2.0 + z)
    t2 = t * t
    return 2.0 * t * (1.0 + t2 * (1.0 / 3.0 + t2 * (0.2 + t2 * (1.0 / 7.0
                      + t2 / 9.0))))

_info = plsc.get_sparse_core_info()
_NS, _L = _info.num_subcores, _info.num_lanes
_NW = _NS


_DN = lax.GatherDimensionNumbers(
    offset_dims=(), collapsed_slice_dims=(0,), start_index_map=(0,))


def _dg(vec, idx):
    return lax.gather(vec, idx[:, None], _DN, slice_sizes=(1,),
                      mode=lax.GatherScatterMode.PROMISE_IN_BOUNDS)


def _lane_bcast(vec, p):
    return _dg(vec, jnp.full((_L,), p, jnp.int32))


def _vsum(vec):
    iot = lax.iota(jnp.int32, _L)
    for sh in (8, 4, 2, 1):
        vec = vec + _dg(vec, iot ^ sh)
    return vec


def _sc_body(n_real, h, w, x_hbm, y_hbm, e_hbm, tab_hbm, parts_hbm, out_hbm,
             x_v, y_v, e_v, qidx_v, cch_v, lan_v, mask_v, rows_v,
             nv_v, dv_v, tmp_v, out_v, sem):
    P = qidx_v.shape[0]
    wid = lax.axis_index("s")
    base = wid * P
    ld0 = pltpu.async_copy(x_hbm.at[pl.ds(base, P)], x_v, sem)
    ld1 = pltpu.async_copy(y_hbm.at[pl.ds(base, P)], y_v, sem)
    ld2 = pltpu.async_copy(e_hbm.at[pl.ds(base, P)], e_v, sem)
    ld0.wait()
    ld1.wait()
    ld2.wait()
    iot = lax.iota(jnp.int32, _L)
    nG = P // _L
    for j in range(nG):
        sl = pl.ds(j * _L, _L)
        x = x_v[sl]
        y = y_v[sl]
        e = e_v[sl]
        n = base + j * _L + iot
        vx = (x >= 0) & (x < w)
        vy = (y >= 0) & (y < h)
        nv = n < n_real
        xs = jnp.where(vx, x, 0)
        ys = jnp.where(vy, y, 0)
        m = jnp.where(vx & vy & nv & (e > 0), 1.0, 0.0).astype(jnp.float32)
        q = jnp.where(nv, n * h + ys, n)
        qidx_v[sl] = q
        cch_v[sl] = xs >> 4
        lan_v[sl] = xs & 15
        mask_v[sl] = m
    half = P // 2
    c0 = pltpu.async_copy(tab_hbm.at[qidx_v.at[pl.ds(0, half)]],
                          rows_v.at[pl.ds(0, half)], sem)
    c1 = pltpu.async_copy(tab_hbm.at[qidx_v.at[pl.ds(half, half)]],
                          rows_v.at[pl.ds(half, half)], sem)
    c0.wait()
    c1.wait()

    num = jnp.zeros((_L,), jnp.float32)
    den = jnp.zeros((_L,), jnp.float32)
    for j in range(nG):
        sl = pl.ds(j * _L, _L)
        cch = cch_v[sl]
        lan = lan_v[sl]
        valg = jnp.zeros((_L,), jnp.float32)
        for p in range(_L):
            pp = j * _L + p
            oh = (1 - jnp.minimum(jnp.abs(iot - _lane_bcast(cch, p)),
                                  1)).astype(jnp.float32)
            acc = jnp.zeros((_L,), jnp.float32)
            for c in range(w // _L):
                acc = acc + rows_v[pp, pl.ds(c * _L, _L)] * oh[c]
            v_b = _dg(acc, _lane_bcast(lan, p))
            eqp = (1 - jnp.minimum(jnp.abs(iot - p), 1)).astype(jnp.float32)
            valg = valg + v_b * eqp
        v = valg
        ea = jnp.exp(-jnp.abs(v))
        sp = jnp.maximum(-v, 0.0) + _log1p(ea)
        z = _EPS * (1.0 + jnp.exp(-jnp.maximum(v, -40.0)))
        loss_main = sp - _log1p(z)
        ww = jnp.exp(jnp.minimum(v, 0.0))
        ratio = ww / ((1.0 + ww) * _EPS)
        loss_tail = -_LNEPS - _log1p(ratio)
        tsel = jnp.maximum(jnp.sign(-13.8 - v), 0.0)
        lossg = loss_tail * tsel + loss_main * (1.0 - tsel)
        mg = mask_v[sl]
        num = num + lossg * mg
        den = den + mg

    nv_v[...] = num
    dv_v[...] = den
    pltpu.sync_copy(nv_v, parts_hbm.at[2 * wid])
    pltpu.sync_copy(dv_v, parts_hbm.at[2 * wid + 1])
    plsc.subcore_barrier()

    @pl.when(wid == 0)
    def _final():
        pltpu.sync_copy(parts_hbm, tmp_v)
        tn = jnp.zeros((_L,), jnp.float32)
        td = jnp.zeros((_L,), jnp.float32)
        for t in range(_NW):
            tn = tn + tmp_v[2 * t]
            td = td + tmp_v[2 * t + 1]
        sn = _vsum(tn)
        sd = _vsum(td)
        out_v[...] = sn / (sd + _EPS)
        pltpu.sync_copy(out_v, out_hbm)


def kernel(feats, xyens):
    B, K, H, W = feats.shape
    N = B * K
    chunk = _NW * _L
    P = ((N + chunk - 1) // chunk) * _L
    NPAD = _NW * P

    xy = xyens.reshape(N, 3).astype(jnp.int32)
    pad = NPAD - N
    x_flat = jnp.pad(xy[:, 0], (0, pad))
    y_flat = jnp.pad(xy[:, 1], (0, pad))
    e_flat = jnp.pad(xy[:, 2], (0, pad))
    tab = feats.reshape(B * K * H, W)

    sc_call = pl.kernel(
        functools.partial(_sc_body, N, H, W),
        mesh=plsc.VectorSubcoreMesh(core_axis_name="c", subcore_axis_name="s",
                                    num_cores=1),
        out_type=[jax.ShapeDtypeStruct((2 * _NW, _L), jnp.float32),
                  jax.ShapeDtypeStruct((_L,), jnp.float32)],
        scratch_types=[
            pltpu.VMEM((P,), jnp.int32),
            pltpu.VMEM((P,), jnp.int32),
            pltpu.VMEM((P,), jnp.int32),
            pltpu.VMEM((P,), jnp.int32),
            pltpu.VMEM((P,), jnp.int32),
            pltpu.VMEM((P,), jnp.int32),
            pltpu.VMEM((P,), jnp.float32),
            pltpu.VMEM((P, W), jnp.float32),
            pltpu.VMEM((_L,), jnp.float32),
            pltpu.VMEM((_L,), jnp.float32),
            pltpu.VMEM((2 * _NW, _L), jnp.float32),
            pltpu.VMEM((_L,), jnp.float32),
            pltpu.SemaphoreType.DMA,
        ],
    )
    _, out = sc_call(x_flat, y_flat, e_flat, tab)
    return out[0]
